# baseline (speedup 1.0000x reference)
import functools

import jax
import jax.numpy as jnp
from jax import lax
from jax.experimental import pallas as pl
from jax.experimental.pallas import tpu as pltpu
from jax.experimental.pallas import tpu_sc as plsc

_POS_W = 2.0
_LANES = 16
_NUM_CORES = 2
_NUM_SUBCORES = 16
_NUNITS = _NUM_CORES * _NUM_SUBCORES
_SC_BLK = 8192


def _sc_partials(o_flat, t_flat):
    n = o_flat.shape[0]
    mesh = plsc.VectorSubcoreMesh(core_axis_name="c", subcore_axis_name="s")

    @functools.partial(
        pl.kernel,
        out_type=jax.ShapeDtypeStruct((_NUNITS, _LANES), jnp.float32),
        mesh=mesh,
        scratch_types=[pltpu.VMEM((_LANES,), jnp.float32)],
    )
    def k(o_hbm, t_hbm, part_hbm, acc_v):
        wid = lax.axis_index("c") * _NUM_SUBCORES + lax.axis_index("s")
        acc_v[...] = jnp.zeros((_LANES,), jnp.float32)

        def body(o_vm, t_vm):
            def it(i, acc):
                o = o_vm[pl.ds(i * _LANES, _LANES)]
                t = t_vm[pl.ds(i * _LANES, _LANES)]
                return acc + jnp.where(
                    t > 0, -_POS_W * o, jnp.where(t < 0, o, 0.0)
                )

            acc_v[...] += lax.fori_loop(
                0, _SC_BLK // _LANES, it, jnp.zeros((_LANES,), jnp.float32)
            )

        pltpu.emit_pipeline(
            body,
            grid=(n // _SC_BLK,),
            in_specs=[
                pl.BlockSpec((_SC_BLK,), lambda i: (i,)),
                pl.BlockSpec((_SC_BLK,), lambda i: (i,)),
            ],
            core_axis_name=("c", "s"),
            dimension_semantics=(pltpu.PARALLEL,),
        )(o_hbm, t_hbm)

        pltpu.sync_copy(acc_v, part_hbm.at[wid])

    return k(o_flat, t_flat)


def kernel(output, target):
    parts = _sc_partials(output.reshape(-1), target.reshape(-1))
    return jnp.sum(parts)


# speedup vs baseline: 1.1857x; 1.1857x over previous
import functools

import jax
import jax.numpy as jnp
from jax import lax
from jax.experimental import pallas as pl
from jax.experimental.pallas import tpu as pltpu
from jax.experimental.pallas import tpu_sc as plsc

_POS_W = 2.0
_LANES = 16
_NUM_CORES = 2
_NUM_SUBCORES = 16
_NUNITS = _NUM_CORES * _NUM_SUBCORES
_SC_BLK = 8192
_UNROLL = 8


def _sc_partials(o_flat, t_flat):
    n = o_flat.shape[0]
    mesh = plsc.VectorSubcoreMesh(core_axis_name="c", subcore_axis_name="s")

    @functools.partial(
        pl.kernel,
        out_type=jax.ShapeDtypeStruct((_NUNITS, _LANES), jnp.float32),
        mesh=mesh,
        scratch_types=[pltpu.VMEM((_LANES,), jnp.float32)],
    )
    def k(o_hbm, t_hbm, part_hbm, acc_v):
        wid = lax.axis_index("c") * _NUM_SUBCORES + lax.axis_index("s")
        acc_v[...] = jnp.zeros((_LANES,), jnp.float32)

        def body(o_vm, t_vm):
            def it(i, accs):
                base = i * (_LANES * _UNROLL)
                new = []
                for u in range(_UNROLL):
                    o = o_vm[pl.ds(base + u * _LANES, _LANES)]
                    t = t_vm[pl.ds(base + u * _LANES, _LANES)]
                    new.append(
                        accs[u]
                        + jnp.where(
                            t > 0, -_POS_W * o, jnp.where(t < 0, o, 0.0)
                        )
                    )
                return tuple(new)

            zero = jnp.zeros((_LANES,), jnp.float32)
            accs = lax.fori_loop(
                0, _SC_BLK // (_LANES * _UNROLL), it, (zero,) * _UNROLL
            )
            total = accs[0]
            for u in range(1, _UNROLL):
                total = total + accs[u]
            acc_v[...] += total

        pltpu.emit_pipeline(
            body,
            grid=(n // _SC_BLK,),
            in_specs=[
                pl.BlockSpec((_SC_BLK,), lambda i: (i,)),
                pl.BlockSpec((_SC_BLK,), lambda i: (i,)),
            ],
            core_axis_name=("c", "s"),
            dimension_semantics=(pltpu.PARALLEL,),
        )(o_hbm, t_hbm)

        pltpu.sync_copy(acc_v, part_hbm.at[wid])

    return k(o_flat, t_flat)


def kernel(output, target):
    parts = _sc_partials(output.reshape(-1), target.reshape(-1))
    return jnp.sum(parts)


# speedup vs baseline: 1.7491x; 1.4751x over previous
import functools

import jax
import jax.numpy as jnp
from jax import lax
from jax.experimental import pallas as pl
from jax.experimental.pallas import tpu as pltpu
from jax.experimental.pallas import tpu_sc as plsc

_POS_W = 2.0
_LANES = 16
_NUM_CORES = 2
_NUM_SUBCORES = 16
_NUNITS = _NUM_CORES * _NUM_SUBCORES
_BLK_R = 8
_BLK_C = 2048
_UNROLL = 8


def _sc_partials(o2d, t2d):
    rows, cols = o2d.shape
    mesh = plsc.VectorSubcoreMesh(core_axis_name="c", subcore_axis_name="s")

    @functools.partial(
        pl.kernel,
        out_type=jax.ShapeDtypeStruct((_NUNITS, _LANES), jnp.float32),
        mesh=mesh,
        scratch_types=[pltpu.VMEM((_LANES,), jnp.float32)],
    )
    def k(o_hbm, t_hbm, part_hbm, acc_v):
        wid = lax.axis_index("c") * _NUM_SUBCORES + lax.axis_index("s")
        acc_v[...] = jnp.zeros((_LANES,), jnp.float32)

        def body(o_vm, t_vm):
            def it(i, accs):
                r = i // (_BLK_C // (_LANES * _UNROLL))
                cb = (i % (_BLK_C // (_LANES * _UNROLL))) * (_LANES * _UNROLL)
                new = []
                for u in range(_UNROLL):
                    o = o_vm[r, pl.ds(cb + u * _LANES, _LANES)]
                    t = t_vm[r, pl.ds(cb + u * _LANES, _LANES)]
                    new.append(
                        accs[u]
                        + jnp.where(
                            t > 0, -_POS_W * o, jnp.where(t < 0, o, 0.0)
                        )
                    )
                return tuple(new)

            zero = jnp.zeros((_LANES,), jnp.float32)
            n_it = (_BLK_R * _BLK_C) // (_LANES * _UNROLL)
            accs = lax.fori_loop(0, n_it, it, (zero,) * _UNROLL)
            total = accs[0]
            for u in range(1, _UNROLL):
                total = total + accs[u]
            acc_v[...] += total

        pltpu.emit_pipeline(
            body,
            grid=(rows // _BLK_R, cols // _BLK_C),
            in_specs=[
                pl.BlockSpec((_BLK_R, _BLK_C), lambda i, j: (i, j)),
                pl.BlockSpec((_BLK_R, _BLK_C), lambda i, j: (i, j)),
            ],
            core_axis_name=("c", "s"),
            dimension_semantics=(pltpu.PARALLEL, pltpu.PARALLEL),
        )(o_hbm, t_hbm)

        pltpu.sync_copy(acc_v, part_hbm.at[wid])

    return k(o2d, t2d)


def kernel(output, target):
    parts = _sc_partials(output, target)
    return jnp.sum(parts)


# speedup vs baseline: 2.4952x; 1.4266x over previous
import functools

import jax
import jax.numpy as jnp
from jax import lax
from jax.experimental import pallas as pl
from jax.experimental.pallas import tpu as pltpu
from jax.experimental.pallas import tpu_sc as plsc

_POS_W = 2.0
_LANES = 16
_NUM_CORES = 2
_NUM_SUBCORES = 16
_NUNITS = _NUM_CORES * _NUM_SUBCORES
_BLK_R = 8
_BLK_C = 2048
_UNROLL = 8
_SC_ROWS = 48
_TC_BLK_R = 16


def _sc_partials(o2d, t2d):
    rows, cols = o2d.shape
    mesh = plsc.VectorSubcoreMesh(core_axis_name="c", subcore_axis_name="s")

    @functools.partial(
        pl.kernel,
        out_type=jax.ShapeDtypeStruct((_NUNITS, _LANES), jnp.float32),
        mesh=mesh,
        scratch_types=[pltpu.VMEM((_LANES,), jnp.float32)],
    )
    def k(o_hbm, t_hbm, part_hbm, acc_v):
        wid = lax.axis_index("c") * _NUM_SUBCORES + lax.axis_index("s")
        acc_v[...] = jnp.zeros((_LANES,), jnp.float32)

        def body(o_vm, t_vm):
            def it(i, accs):
                r = i // (_BLK_C // (_LANES * _UNROLL))
                cb = (i % (_BLK_C // (_LANES * _UNROLL))) * (_LANES * _UNROLL)
                new = []
                for u in range(_UNROLL):
                    o = o_vm[r, pl.ds(cb + u * _LANES, _LANES)]
                    t = t_vm[r, pl.ds(cb + u * _LANES, _LANES)]
                    new.append(
                        accs[u]
                        + jnp.where(
                            t > 0, -_POS_W * o, jnp.where(t < 0, o, 0.0)
                        )
                    )
                return tuple(new)

            zero = jnp.zeros((_LANES,), jnp.float32)
            n_it = (_BLK_R * _BLK_C) // (_LANES * _UNROLL)
            accs = lax.fori_loop(0, n_it, it, (zero,) * _UNROLL)
            total = accs[0]
            for u in range(1, _UNROLL):
                total = total + accs[u]
            acc_v[...] += total

        ncol = cols // _BLK_C
        pltpu.emit_pipeline(
            body,
            grid=((_SC_ROWS // _BLK_R) * ncol,),
            in_specs=[
                pl.BlockSpec(
                    (_BLK_R, _BLK_C), lambda i: (i // ncol, i % ncol)
                ),
                pl.BlockSpec(
                    (_BLK_R, _BLK_C), lambda i: (i // ncol, i % ncol)
                ),
            ],
            core_axis_name=("c", "s"),
            dimension_semantics=(pltpu.PARALLEL,),
        )(o_hbm, t_hbm)

        pltpu.sync_copy(acc_v, part_hbm.at[wid])

    return k(o2d, t2d)


def _tc_body(out_ref, tgt_ref, acc_ref):
    i = pl.program_id(0)
    o = out_ref[...]
    t = tgt_ref[...]
    w = jnp.where(t > 0, -_POS_W * o, jnp.where(t < 0, o, 0.0))
    p = jnp.sum(w)

    @pl.when(i == 0)
    def _():
        acc_ref[0, 0] = 0.0

    acc_ref[0, 0] += p


def _tc_partial(output, target, row_start):
    rows, cols = output.shape
    n_blocks = (rows - row_start) // _TC_BLK_R
    blk_start = row_start // _TC_BLK_R
    res = pl.pallas_call(
        _tc_body,
        grid=(n_blocks,),
        in_specs=[
            pl.BlockSpec((_TC_BLK_R, cols), lambda i: (i + blk_start, 0)),
            pl.BlockSpec((_TC_BLK_R, cols), lambda i: (i + blk_start, 0)),
        ],
        out_specs=pl.BlockSpec(
            (1, 1), lambda i: (0, 0), memory_space=pltpu.SMEM
        ),
        out_shape=jax.ShapeDtypeStruct((1, 1), jnp.float32),
    )(output, target)
    return res[0, 0]


def kernel(output, target):
    parts = _sc_partials(output, target)
    tc = _tc_partial(output, target, _SC_ROWS)
    return tc + jnp.sum(parts)


# speedup vs baseline: 4.6921x; 1.8804x over previous
import jax
import jax.numpy as jnp
from jax.experimental import pallas as pl
from jax.experimental.pallas import tpu as pltpu

_POS_W = 2.0
_BLOCK_ROWS = 8


def _reduce_body(out_ref, tgt_ref, acc_ref):
    i = pl.program_id(0)
    o = out_ref[...]
    t = tgt_ref[...]
    w = jnp.where(t > 0, -_POS_W * o, jnp.where(t < 0, o, 0.0))
    p = jnp.sum(w)

    @pl.when(i == 0)
    def _():
        acc_ref[0, 0] = 0.0

    acc_ref[0, 0] += p


def kernel(output, target):
    rows, cols = output.shape
    res = pl.pallas_call(
        _reduce_body,
        grid=(rows // _BLOCK_ROWS,),
        in_specs=[
            pl.BlockSpec((_BLOCK_ROWS, cols), lambda i: (i, 0)),
            pl.BlockSpec((_BLOCK_ROWS, cols), lambda i: (i, 0)),
        ],
        out_specs=pl.BlockSpec(
            (1, 1), lambda i: (0, 0), memory_space=pltpu.SMEM
        ),
        out_shape=jax.ShapeDtypeStruct((1, 1), jnp.float32),
    )(output, target)
    return res[0, 0]


# speedup vs baseline: 6.6284x; 1.4127x over previous
import jax
import jax.numpy as jnp
from jax.experimental import pallas as pl
from jax.experimental.pallas import tpu as pltpu

_POS_W = 2.0
_BLOCK_ROWS = 32


def _reduce_body(out_ref, tgt_ref, acc_ref):
    i = pl.program_id(0)
    o = out_ref[...]
    t = tgt_ref[...]
    w = jnp.where(t > 0, -_POS_W * o, jnp.where(t < 0, o, 0.0))
    p = jnp.sum(w)

    @pl.when(i == 0)
    def _():
        acc_ref[0, 0] = 0.0

    acc_ref[0, 0] += p


def kernel(output, target):
    rows, cols = output.shape
    res = pl.pallas_call(
        _reduce_body,
        grid=(rows // _BLOCK_ROWS,),
        in_specs=[
            pl.BlockSpec((_BLOCK_ROWS, cols), lambda i: (i, 0)),
            pl.BlockSpec((_BLOCK_ROWS, cols), lambda i: (i, 0)),
        ],
        out_specs=pl.BlockSpec(
            (1, 1), lambda i: (0, 0), memory_space=pltpu.SMEM
        ),
        out_shape=jax.ShapeDtypeStruct((1, 1), jnp.float32),
    )(output, target)
    return res[0, 0]
